# baseline (speedup 1.0000x reference)
import functools
import jax
import jax.numpy as jnp
from jax import lax
from jax.experimental import pallas as pl
from jax.experimental.pallas import tpu as pltpu
from jax.experimental.pallas import tpu_sc as plsc

N = 10000
D = 256
NE = 80000
NREL = 401
MBLK = 2000
GRID = N // MBLK

NC = 2
NS = 16
L = 16
EP = 81920
EPT = EP // NS
NCH = EPT // 128
DUMN = 10008
NPAD = 10240
ACCR = 10240
QN = 2500
CSPQ = 1015808
CQUSE = QN * NREL
CDUM = CQUSE + 4


def _tc_prep_body(x_ref, w3_ref, rel_ref, wrel_ref, dinvT_ref,
                  y_in_ref, y_out_ref, xwl_ref, rw3_ref, rnext_ref):
    i = pl.program_id(0)
    xb = x_ref[...]
    w3 = w3_ref[...]
    rel = rel_ref[...]
    xw3 = jnp.dot(xb, w3, preferred_element_type=jnp.float32)
    loopvec = jnp.dot(rel[NREL - 1:], w3[:, 2 * D:], preferred_element_type=jnp.float32)
    di = dinvT_ref[:, 0:1]
    do = dinvT_ref[:, 1:2]
    y_in = di * xw3[:, :D]
    y_out = do * xw3[:, D:2 * D]
    y_in_ref[0] = y_in[:, :128]
    y_in_ref[1] = y_in[:, 128:]
    y_out_ref[0] = y_out[:, :128]
    y_out_ref[1] = y_out[:, 128:]
    xwl_ref[...] = xw3[:, 2 * D:] - loopvec

    @pl.when(i == 0)
    def _():
        rw3_ref[...] = jnp.dot(rel, w3[:, :2 * D],
                               preferred_element_type=jnp.float32)
        rnext_ref[...] = jnp.dot(rel[:NREL - 1], wrel_ref[...],
                                 preferred_element_type=jnp.float32)


def _tc_prep(x, w3, rel_all, w_rel, dinvT):
    return pl.pallas_call(
        _tc_prep_body,
        grid=(GRID,),
        in_specs=[
            pl.BlockSpec((MBLK, D), lambda i: (i, 0)),
            pl.BlockSpec((D, 3 * D), lambda i: (0, 0)),
            pl.BlockSpec((NREL, D), lambda i: (0, 0)),
            pl.BlockSpec((D, D), lambda i: (0, 0)),
            pl.BlockSpec((MBLK, 2), lambda i: (i, 0)),
        ],
        out_specs=[
            pl.BlockSpec((2, MBLK, 128), lambda i: (0, i, 0)),
            pl.BlockSpec((2, MBLK, 128), lambda i: (0, i, 0)),
            pl.BlockSpec((MBLK, D), lambda i: (i, 0)),
            pl.BlockSpec((NREL, 2 * D), lambda i: (0, 0)),
            pl.BlockSpec((NREL - 1, D), lambda i: (0, 0)),
        ],
        out_shape=[
            jax.ShapeDtypeStruct((2, N, 128), jnp.float32),
            jax.ShapeDtypeStruct((2, N, 128), jnp.float32),
            jax.ShapeDtypeStruct((N, D), jnp.float32),
            jax.ShapeDtypeStruct((NREL, 2 * D), jnp.float32),
            jax.ShapeDtypeStruct((NREL - 1, D), jnp.float32),
        ],
    )(x, w3, rel_all, w_rel, dinvT)


def _tc_combine_body(acc_ref, cin_ref, cout_ref, rw3_ref, xwl_ref,
                     dinvT_ref, pre_ref, stat_ref):
    i = pl.program_id(0)
    di = dinvT_ref[:, 0:1]
    do = dinvT_ref[:, 1:2]
    rw3 = rw3_ref[...]
    acc_in = jnp.concatenate([acc_ref[0, 0], acc_ref[0, 1]], axis=1)
    acc_out = jnp.concatenate([acc_ref[1, 0], acc_ref[1, 1]], axis=1)
    rin = jnp.dot(cin_ref[...], rw3[:, :D], preferred_element_type=jnp.float32)
    rout = jnp.dot(cout_ref[...], rw3[:, D:], preferred_element_type=jnp.float32)
    pre = (di * (acc_in - rin) + do * (acc_out - rout)
           + xwl_ref[...]) * (1.0 / 3.0)
    pre_ref[...] = pre
    s = jnp.sum(pre, axis=0, keepdims=True)
    s2 = jnp.sum(pre * pre, axis=0, keepdims=True)
    blk = jnp.concatenate([s, s2], axis=0)

    @pl.when(i == 0)
    def _():
        stat_ref[...] = blk

    @pl.when(i > 0)
    def _():
        stat_ref[...] = stat_ref[...] + blk


def _tc_combine(acc4, c_in, c_out, rw3, xwl, dinvT):
    return pl.pallas_call(
        _tc_combine_body,
        grid=(GRID,),
        in_specs=[
            pl.BlockSpec((2, 2, MBLK, 128), lambda i: (0, 0, i, 0)),
            pl.BlockSpec((MBLK, NREL), lambda i: (i, 0)),
            pl.BlockSpec((MBLK, NREL), lambda i: (i, 0)),
            pl.BlockSpec((NREL, 2 * D), lambda i: (0, 0)),
            pl.BlockSpec((MBLK, D), lambda i: (i, 0)),
            pl.BlockSpec((MBLK, 2), lambda i: (i, 0)),
        ],
        out_specs=[
            pl.BlockSpec((MBLK, D), lambda i: (i, 0)),
            pl.BlockSpec((2, D), lambda i: (0, 0)),
        ],
        out_shape=[
            jax.ShapeDtypeStruct((N, D), jnp.float32),
            jax.ShapeDtypeStruct((2, D), jnp.float32),
        ],
    )(acc4, c_in, c_out, rw3, xwl, dinvT)


def _tc_bn_body(pre_ref, stat_ref, out_ref):
    s = stat_ref[...]
    mean = s[0:1] * (1.0 / N)
    var = s[1:2] * (1.0 / N) - mean * mean
    inv = lax.rsqrt(var + 1e-5)
    out_ref[...] = jnp.maximum((pre_ref[...] - mean) * inv, 0.0)


def _tc_bn(pre, stat):
    return pl.pallas_call(
        _tc_bn_body,
        grid=(GRID,),
        in_specs=[
            pl.BlockSpec((MBLK, D), lambda i: (i, 0)),
            pl.BlockSpec((2, D), lambda i: (0, 0)),
        ],
        out_specs=pl.BlockSpec((MBLK, D), lambda i: (i, 0)),
        out_shape=jax.ShapeDtypeStruct((N, D), jnp.float32),
    )(pre, stat)


def _tc_dinv_body(deg_ref, out_ref):
    d = deg_ref[...]
    out_ref[...] = jnp.where(d > 0, d ** -0.5, 0.0)


def _tc_dinv(deg2):
    return pl.pallas_call(
        _tc_dinv_body,
        out_shape=jax.ShapeDtypeStruct(deg2.shape, jnp.float32),
    )(deg2)


_MESH = plsc.VectorSubcoreMesh(core_axis_name="c", subcore_axis_name="s")


def _sc_deg_body(srcd_ref, deg_ref, acc_sp, idxb, ones, zb):
    c = lax.axis_index("c")
    s = lax.axis_index("s")
    for k in range(40):
        zb[pl.ds(L * k, L)] = jnp.zeros((L,), jnp.float32)
    for k in range(8):
        ones[pl.ds(L * k, L)] = jnp.full((L,), 1.0, jnp.float32)
    pltpu.sync_copy(zb, acc_sp.at[pl.ds(640 * s, 640)])
    plsc.subcore_barrier()
    pltpu.sync_copy(srcd_ref.at[c, s], idxb)

    @pl.loop(0, NCH)
    def _(j):
        pltpu.sync_copy(ones, acc_sp.at[idxb.at[j]], add=True)

    plsc.subcore_barrier()
    pltpu.sync_copy(acc_sp.at[pl.ds(640 * s, 640)], zb)
    pltpu.sync_copy(zb, deg_ref.at[pl.ds(c * NPAD + 640 * s, 640)])


def _sc_deg(srcd):
    return pl.kernel(
        _sc_deg_body,
        out_type=jax.ShapeDtypeStruct((NC * NPAD,), jnp.float32),
        mesh=_MESH,
        scratch_types=[
            pltpu.VMEM_SHARED((NPAD,), jnp.float32),
            pltpu.VMEM((NCH, 128), jnp.int32),
            pltpu.VMEM((128,), jnp.float32),
            pltpu.VMEM((640,), jnp.float32),
        ],
    )(srcd)


def _sc_cprime_body(dstw_ref, etp_ref, srcd_ref, dinv_ref, cfl_ref,
                    csp, dstb, etb, srcb, valb, sidx, zb, bb,
                    gsem, ssems, osems, zsem):
    c = lax.axis_index("c")
    s = lax.axis_index("s")
    for k in range(128):
        zb[pl.ds(L * k, L)] = jnp.zeros((L,), jnp.float32)
    for d in range(2):
        pltpu.sync_copy(dstw_ref.at[d, s], dstb)
        pltpu.sync_copy(etp_ref.at[d, s], etb)
        pltpu.sync_copy(srcd_ref.at[d, s], srcb)

        @pl.loop(0, NCH)
        def _(j):
            for k in range(8):
                sl = pl.ds(L * k, L)
                srcb.at[j][sl] = srcb.at[j][sl] + d * NPAD

        for j0 in range(0, NCH, 8):
            gds = [pltpu.async_copy(dinv_ref.at[srcb.at[j]], valb.at[j],
                                    gsem) for j in range(j0, j0 + 8)]
            for cp in gds:
                cp.wait()

        for q2 in range(2):
            quarter = 2 * q2 + c
            base = quarter * QN

            for k0 in range(0, 31, 8):
                zd = [pltpu.async_copy(
                    zb, csp.at[pl.ds(2048 * (31 * s + k), 2048)], zsem)
                    for k in range(k0, min(k0 + 8, 31))]
                for cp in zd:
                    cp.wait()
            plsc.subcore_barrier()

            def stage(jj, b):
                for k in range(8):
                    sl = pl.ds(L * k, L)
                    dst16 = dstb.at[jj][sl]
                    et16 = etb.at[jj][sl]
                    m = (dst16 >= base) & (dst16 < base + QN)
                    flat = jnp.where(m, (dst16 - base) * NREL + et16, CDUM)
                    sidx[b][sl] = flat

            def s_issue(jj, b):
                pltpu.async_copy(valb.at[jj], csp.at[sidx[b]], ssems[b],
                                 add=True)

            for b in range(4):
                stage(b, b)

            @pl.loop(0, NCH, step=4)
            def _(j):
                for b in range(4):
                    jj = j + b
                    s_issue(jj, b)
                    pltpu.make_async_copy(valb.at[jj], csp.at[sidx[b]],
                                          ssems[b]).wait()

                    @pl.when(jj + 4 < NCH)
                    def _():
                        stage(jj + 4, b)

            plsc.subcore_barrier()
            hbase = (d * 4 + quarter) * CSPQ
            od = {}
            for k in range(31):
                b = k % 2
                if k - 2 >= 0:
                    od[k - 2].wait()
                off = 63488 * s + 2048 * k
                pltpu.sync_copy(csp.at[pl.ds(off, 2048)], bb[b])
                od[k] = pltpu.async_copy(
                    bb[b], cfl_ref.at[pl.ds(hbase + off, 2048)], osems[b])
            od[29].wait()
            od[30].wait()
            plsc.subcore_barrier()


def _sc_cprime(dstw, etp, srcd, dinv2):
    return pl.kernel(
        _sc_cprime_body,
        out_type=jax.ShapeDtypeStruct((8 * CSPQ,), jnp.float32),
        mesh=_MESH,
        scratch_types=[
            pltpu.VMEM_SHARED((CSPQ,), jnp.float32),
            pltpu.VMEM((NCH, 128), jnp.int32),
            pltpu.VMEM((NCH, 128), jnp.int32),
            pltpu.VMEM((NCH, 128), jnp.int32),
            pltpu.VMEM((NCH, 128), jnp.float32),
            [pltpu.VMEM((128,), jnp.int32) for _ in range(4)],
            pltpu.VMEM((2048,), jnp.float32),
            [pltpu.VMEM((2048,), jnp.float32) for _ in range(2)],
            pltpu.SemaphoreType.DMA,
            [pltpu.SemaphoreType.DMA for _ in range(4)],
            [pltpu.SemaphoreType.DMA for _ in range(2)],
            pltpu.SemaphoreType.DMA,
        ],
    )(dstw, etp, srcd, dinv2)


def _sc_acc_body(yin_ref, yout_ref, srci_ref, dsti_ref, acc4_ref,
                 accsp, srcb, dstb, gi, wi, rb, zb, gsems, ssems, zsem):
    c = lax.axis_index("c")
    s = lax.axis_index("s")
    roff = c * N
    CH = 64
    NCHV = EPT // CH
    for r in range(L):
        for k in range(8):
            zb[r, pl.ds(L * k, L)] = jnp.zeros((L,), jnp.float32)
    for d in range(2):
        ytab = yin_ref if d == 0 else yout_ref

        for k0 in range(0, 40, 8):
            zd = [pltpu.async_copy(zb, accsp.at[pl.ds(640 * s + L * k, L)],
                                   zsem) for k in range(k0, k0 + 8)]
            for cp in zd:
                cp.wait()
        plsc.subcore_barrier()
        pltpu.sync_copy(srci_ref.at[d, pl.ds(EPT * s, EPT)], srcb)
        pltpu.sync_copy(dsti_ref.at[d, pl.ds(EPT * s, EPT)], dstb)

        def stage(jj, b):
            for k in range(CH // L):
                sl = pl.ds(L * k, L)
                gi[b][sl] = srcb[pl.ds(CH * jj + L * k, L)] + roff
                wi[b][sl] = dstb[pl.ds(CH * jj + L * k, L)]

        for b in range(4):
            stage(b, b)
            pltpu.async_copy(ytab.at[gi[b]], rb[b], gsems[b])

        @pl.loop(0, NCHV, step=4)
        def _(j):
            for b in range(4):
                jj = j + b
                pltpu.make_async_copy(ytab.at[gi[b]], rb[b],
                                      gsems[b]).wait()
                pltpu.async_copy(rb[b], accsp.at[wi[b]], ssems[b],
                                 add=True)
                pltpu.make_async_copy(rb[b], accsp.at[wi[b]],
                                      ssems[b]).wait()

                @pl.when(jj + 4 < NCHV)
                def _():
                    stage(jj + 4, b)
                    pltpu.async_copy(ytab.at[gi[b]], rb[b], gsems[b])

        plsc.subcore_barrier()
        od = {}
        for k in range(10):
            b = k % 4
            if k - 4 >= 0:
                od[k - 4].wait()
            r0 = 640 * s + 64 * k
            pltpu.sync_copy(accsp.at[pl.ds(r0, 64)], rb[b])
            od[k] = pltpu.async_copy(
                rb[b], acc4_ref.at[d, c, pl.ds(r0, 64)], gsems[b])
        for k in range(6, 10):
            od[k].wait()
        plsc.subcore_barrier()


def _sc_acc(yin2, yout2, srcg, dstw):
    return pl.kernel(
        _sc_acc_body,
        out_type=jax.ShapeDtypeStruct((2, NC, ACCR, 128), jnp.float32),
        mesh=_MESH,
        scratch_types=[
            pltpu.VMEM_SHARED((ACCR, 128), jnp.float32),
            pltpu.VMEM((EPT,), jnp.int32),
            pltpu.VMEM((EPT,), jnp.int32),
            [pltpu.VMEM((64,), jnp.int32) for _ in range(4)],
            [pltpu.VMEM((64,), jnp.int32) for _ in range(4)],
            [pltpu.VMEM((64, 128), jnp.float32) for _ in range(4)],
            pltpu.VMEM((L, 128), jnp.float32),
            [pltpu.SemaphoreType.DMA for _ in range(4)],
            [pltpu.SemaphoreType.DMA for _ in range(4)],
            pltpu.SemaphoreType.DMA,
        ],
    )(yin2, yout2, srcg, dstw)


def kernel(x, rels, edge_index, edge_type,
           w_in1, w_out1, w_loop1, w_rel1,
           w_in2, w_out2, w_loop2, w_rel2,
           loop_rel1, loop_rel2):
    ne = edge_index.shape[1] // 2
    src_in, dst_in = edge_index[0, :ne], edge_index[1, :ne]
    src_out, dst_out = edge_index[0, ne:], edge_index[1, ne:]
    et_in, et_out = edge_type[:ne], edge_type[ne:]

    npad = EP - ne
    pad0 = jnp.zeros((npad,), jnp.int32)
    padd = jnp.full((npad,), DUMN, jnp.int32)

    def lay(a_in, a_out, pad):
        return jnp.stack([jnp.concatenate([a_in, pad]),
                          jnp.concatenate([a_out, pad])]).reshape(2, NS, NCH, 128)

    srcg = lay(src_in, src_out, pad0)
    srcd = lay(src_in, src_out, padd)
    dstw = lay(dst_in, dst_out, padd)
    etp = lay(et_in, et_out, pad0)

    deg2 = _sc_deg(srcd).reshape(NC, NPAD)
    dinv2 = _tc_dinv(deg2)
    dinvT = dinv2[:, :N].T
    cfl = _sc_cprime(dstw, etp, srcd, dinv2.reshape(-1))

    def cq(d, q):
        b = (d * 4 + q) * CSPQ
        return lax.dynamic_slice(cfl, (b,), (CQUSE,)).reshape(QN, NREL)

    c_in = jnp.concatenate([cq(0, q) for q in range(4)], axis=0)
    c_out = jnp.concatenate([cq(1, q) for q in range(4)], axis=0)

    w3_1 = jnp.concatenate([w_in1, w_out1, w_loop1], axis=1)
    w3_2 = jnp.concatenate([w_in2, w_out2, w_loop2], axis=1)
    rel_all1 = jnp.concatenate([rels, loop_rel1], axis=0)

    def layer(xl, rel_all, w3, w_rel):
        y_in, y_out, xwl, rw3, rnext = _tc_prep(
            xl, w3, rel_all, w_rel, dinvT)
        acc4 = _sc_acc(y_in.reshape(2 * N, 128), y_out.reshape(2 * N, 128),
                       srcg.reshape(2, EP), dstw.reshape(2, EP))
        pre, stat = _tc_combine(acc4, c_in, c_out, rw3, xwl, dinvT)
        return _tc_bn(pre, stat), rnext

    x1, r1 = layer(x, rel_all1, w3_1, w_rel1)
    rel_all2 = jnp.concatenate([r1, loop_rel2], axis=0)
    x2, r2 = layer(x1, rel_all2, w3_2, w_rel2)
    return (x2, r2)


# speedup vs baseline: 1.0606x; 1.0606x over previous
import functools
import jax
import jax.numpy as jnp
from jax import lax
from jax.experimental import pallas as pl
from jax.experimental.pallas import tpu as pltpu
from jax.experimental.pallas import tpu_sc as plsc

N = 10000
D = 256
NE = 80000
NREL = 401
MBLK = 2000
GRID = N // MBLK

NC = 2
NS = 16
L = 16
EP = 81920
EPT = EP // NS
NCH = EPT // 128
DUMN = 10008
NPAD = 10240
ACCR = 10240
QN = 2500
CSPQ = 1015808
CQUSE = QN * NREL
CDUM = CQUSE + 4


def _tc_prep_body(x_ref, w3_ref, rel_ref, wrel_ref, dinvT_ref,
                  y_in_ref, y_out_ref, xwl_ref, rw3_ref, rnext_ref):
    i = pl.program_id(0)
    xb = x_ref[...]
    w3 = w3_ref[...]
    rel = rel_ref[...]
    xw3 = jnp.dot(xb, w3, preferred_element_type=jnp.float32)
    loopvec = jnp.dot(rel[NREL - 1:], w3[:, 2 * D:], preferred_element_type=jnp.float32)
    di = dinvT_ref[:, 0:1]
    do = dinvT_ref[:, 1:2]
    y_in = di * xw3[:, :D]
    y_out = do * xw3[:, D:2 * D]
    y_in_ref[0] = y_in[:, :128]
    y_in_ref[1] = y_in[:, 128:]
    y_out_ref[0] = y_out[:, :128]
    y_out_ref[1] = y_out[:, 128:]
    xwl_ref[...] = xw3[:, 2 * D:] - loopvec

    @pl.when(i == 0)
    def _():
        rw3_ref[...] = jnp.dot(rel, w3[:, :2 * D],
                               preferred_element_type=jnp.float32)
        rnext_ref[...] = jnp.dot(rel[:NREL - 1], wrel_ref[...],
                                 preferred_element_type=jnp.float32)


def _tc_prep(x, w3, rel_all, w_rel, dinvT):
    return pl.pallas_call(
        _tc_prep_body,
        grid=(GRID,),
        in_specs=[
            pl.BlockSpec((MBLK, D), lambda i: (i, 0)),
            pl.BlockSpec((D, 3 * D), lambda i: (0, 0)),
            pl.BlockSpec((NREL, D), lambda i: (0, 0)),
            pl.BlockSpec((D, D), lambda i: (0, 0)),
            pl.BlockSpec((MBLK, 2), lambda i: (i, 0)),
        ],
        out_specs=[
            pl.BlockSpec((2, MBLK, 128), lambda i: (0, i, 0)),
            pl.BlockSpec((2, MBLK, 128), lambda i: (0, i, 0)),
            pl.BlockSpec((MBLK, D), lambda i: (i, 0)),
            pl.BlockSpec((NREL, 2 * D), lambda i: (0, 0)),
            pl.BlockSpec((NREL - 1, D), lambda i: (0, 0)),
        ],
        out_shape=[
            jax.ShapeDtypeStruct((2, N, 128), jnp.float32),
            jax.ShapeDtypeStruct((2, N, 128), jnp.float32),
            jax.ShapeDtypeStruct((N, D), jnp.float32),
            jax.ShapeDtypeStruct((NREL, 2 * D), jnp.float32),
            jax.ShapeDtypeStruct((NREL - 1, D), jnp.float32),
        ],
    )(x, w3, rel_all, w_rel, dinvT)


def _tc_combine_body(acc_ref, cin_ref, cout_ref, rw3_ref, xwl_ref,
                     dinvT_ref, pre_ref, stat_ref):
    i = pl.program_id(0)
    di = dinvT_ref[:, 0:1]
    do = dinvT_ref[:, 1:2]
    rw3 = rw3_ref[...]
    acc_in = jnp.concatenate([acc_ref[0, 0], acc_ref[0, 1]], axis=1)
    acc_out = jnp.concatenate([acc_ref[1, 0], acc_ref[1, 1]], axis=1)
    rin = jnp.dot(cin_ref[...], rw3[:, :D], preferred_element_type=jnp.float32)
    rout = jnp.dot(cout_ref[...], rw3[:, D:], preferred_element_type=jnp.float32)
    pre = (di * (acc_in - rin) + do * (acc_out - rout)
           + xwl_ref[...]) * (1.0 / 3.0)
    pre_ref[...] = pre
    s = jnp.sum(pre, axis=0, keepdims=True)
    s2 = jnp.sum(pre * pre, axis=0, keepdims=True)
    blk = jnp.concatenate([s, s2], axis=0)

    @pl.when(i == 0)
    def _():
        stat_ref[...] = blk

    @pl.when(i > 0)
    def _():
        stat_ref[...] = stat_ref[...] + blk


def _tc_combine(acc4, c_in, c_out, rw3, xwl, dinvT):
    return pl.pallas_call(
        _tc_combine_body,
        grid=(GRID,),
        in_specs=[
            pl.BlockSpec((2, 2, MBLK, 128), lambda i: (0, 0, i, 0)),
            pl.BlockSpec((MBLK, NREL), lambda i: (i, 0)),
            pl.BlockSpec((MBLK, NREL), lambda i: (i, 0)),
            pl.BlockSpec((NREL, 2 * D), lambda i: (0, 0)),
            pl.BlockSpec((MBLK, D), lambda i: (i, 0)),
            pl.BlockSpec((MBLK, 2), lambda i: (i, 0)),
        ],
        out_specs=[
            pl.BlockSpec((MBLK, D), lambda i: (i, 0)),
            pl.BlockSpec((2, D), lambda i: (0, 0)),
        ],
        out_shape=[
            jax.ShapeDtypeStruct((N, D), jnp.float32),
            jax.ShapeDtypeStruct((2, D), jnp.float32),
        ],
    )(acc4, c_in, c_out, rw3, xwl, dinvT)


def _tc_bn_body(pre_ref, stat_ref, out_ref):
    s = stat_ref[...]
    mean = s[0:1] * (1.0 / N)
    var = s[1:2] * (1.0 / N) - mean * mean
    inv = lax.rsqrt(var + 1e-5)
    out_ref[...] = jnp.maximum((pre_ref[...] - mean) * inv, 0.0)


def _tc_bn(pre, stat):
    return pl.pallas_call(
        _tc_bn_body,
        grid=(GRID,),
        in_specs=[
            pl.BlockSpec((MBLK, D), lambda i: (i, 0)),
            pl.BlockSpec((2, D), lambda i: (0, 0)),
        ],
        out_specs=pl.BlockSpec((MBLK, D), lambda i: (i, 0)),
        out_shape=jax.ShapeDtypeStruct((N, D), jnp.float32),
    )(pre, stat)


def _tc_dinv_body(deg_ref, out_ref):
    d = deg_ref[...]
    out_ref[...] = jnp.where(d > 0, d ** -0.5, 0.0)


def _tc_dinv(deg2):
    return pl.pallas_call(
        _tc_dinv_body,
        out_shape=jax.ShapeDtypeStruct(deg2.shape, jnp.float32),
    )(deg2)


_MESH = plsc.VectorSubcoreMesh(core_axis_name="c", subcore_axis_name="s")


def _sc_deg_body(srcd_ref, deg_ref, acc_sp, idxb, ones, zb):
    c = lax.axis_index("c")
    s = lax.axis_index("s")
    for k in range(40):
        zb[pl.ds(L * k, L)] = jnp.zeros((L,), jnp.float32)
    for k in range(8):
        ones[pl.ds(L * k, L)] = jnp.full((L,), 1.0, jnp.float32)
    pltpu.sync_copy(zb, acc_sp.at[pl.ds(640 * s, 640)])
    plsc.subcore_barrier()
    pltpu.sync_copy(srcd_ref.at[c, s], idxb)

    @pl.loop(0, NCH)
    def _(j):
        pltpu.sync_copy(ones, acc_sp.at[idxb.at[j]], add=True)

    plsc.subcore_barrier()
    pltpu.sync_copy(acc_sp.at[pl.ds(640 * s, 640)], zb)
    pltpu.sync_copy(zb, deg_ref.at[pl.ds(c * NPAD + 640 * s, 640)])


def _sc_deg(srcd):
    return pl.kernel(
        _sc_deg_body,
        out_type=jax.ShapeDtypeStruct((NC * NPAD,), jnp.float32),
        mesh=_MESH,
        scratch_types=[
            pltpu.VMEM_SHARED((NPAD,), jnp.float32),
            pltpu.VMEM((NCH, 128), jnp.int32),
            pltpu.VMEM((128,), jnp.float32),
            pltpu.VMEM((640,), jnp.float32),
        ],
    )(srcd)


def _sc_cprime_body(dstw_ref, etp_ref, srcd_ref, dinv_ref, cfl_ref,
                    csp, dstb, etb, srcb, sidx, gidx, sval, zb, bb,
                    gsems, ssems, osems, zsem):
    c = lax.axis_index("c")
    s = lax.axis_index("s")
    for k in range(128):
        zb[pl.ds(L * k, L)] = jnp.zeros((L,), jnp.float32)
    for d in range(2):
        pltpu.sync_copy(dstw_ref.at[d, s], dstb)
        pltpu.sync_copy(etp_ref.at[d, s], etb)
        pltpu.sync_copy(srcd_ref.at[d, s], srcb)
        for q2 in range(2):
            quarter = 2 * q2 + c
            base = quarter * QN

            for k0 in range(0, 31, 8):
                zd = [pltpu.async_copy(
                    zb, csp.at[pl.ds(2048 * (31 * s + k), 2048)], zsem)
                    for k in range(k0, min(k0 + 8, 31))]
                for cp in zd:
                    cp.wait()
            plsc.subcore_barrier()

            def stage(jj, b):
                for k in range(8):
                    sl = pl.ds(L * k, L)
                    dst16 = dstb.at[jj][sl]
                    et16 = etb.at[jj][sl]
                    src16 = srcb.at[jj][sl]
                    m = (dst16 >= base) & (dst16 < base + QN)
                    flat = jnp.where(m, (dst16 - base) * NREL + et16, CDUM)
                    sidx[b][sl] = flat
                    gidx[b][sl] = src16 + d * NPAD

            for b in range(4):
                stage(b, b)
                pltpu.async_copy(dinv_ref.at[gidx[b]], sval[b], gsems[b])

            @pl.loop(0, NCH, step=4)
            def _(j):
                for b in range(4):
                    jj = j + b
                    pltpu.make_async_copy(dinv_ref.at[gidx[b]], sval[b],
                                          gsems[b]).wait()
                    pltpu.async_copy(sval[b], csp.at[sidx[b]], ssems[b],
                                     add=True)
                    pltpu.make_async_copy(sval[b], csp.at[sidx[b]],
                                          ssems[b]).wait()

                    @pl.when(jj + 4 < NCH)
                    def _():
                        stage(jj + 4, b)
                        pltpu.async_copy(dinv_ref.at[gidx[b]], sval[b],
                                         gsems[b])

            plsc.subcore_barrier()
            hbase = (d * 4 + quarter) * CSPQ
            od = {}
            for k in range(31):
                b = k % 2
                if k - 2 >= 0:
                    od[k - 2].wait()
                off = 63488 * s + 2048 * k
                pltpu.sync_copy(csp.at[pl.ds(off, 2048)], bb[b])
                od[k] = pltpu.async_copy(
                    bb[b], cfl_ref.at[pl.ds(hbase + off, 2048)], osems[b])
            od[29].wait()
            od[30].wait()
            plsc.subcore_barrier()


def _sc_cprime(dstw, etp, srcd, dinv2):
    return pl.kernel(
        _sc_cprime_body,
        out_type=jax.ShapeDtypeStruct((8 * CSPQ,), jnp.float32),
        mesh=_MESH,
        scratch_types=[
            pltpu.VMEM_SHARED((CSPQ,), jnp.float32),
            pltpu.VMEM((NCH, 128), jnp.int32),
            pltpu.VMEM((NCH, 128), jnp.int32),
            pltpu.VMEM((NCH, 128), jnp.int32),
            [pltpu.VMEM((128,), jnp.int32) for _ in range(4)],
            [pltpu.VMEM((128,), jnp.int32) for _ in range(4)],
            [pltpu.VMEM((128,), jnp.float32) for _ in range(4)],
            pltpu.VMEM((2048,), jnp.float32),
            [pltpu.VMEM((2048,), jnp.float32) for _ in range(2)],
            [pltpu.SemaphoreType.DMA for _ in range(4)],
            [pltpu.SemaphoreType.DMA for _ in range(4)],
            [pltpu.SemaphoreType.DMA for _ in range(2)],
            pltpu.SemaphoreType.DMA,
        ],
    )(dstw, etp, srcd, dinv2)


def _sc_acc_body(yin_ref, yout_ref, srci_ref, dsti_ref, acc4_ref,
                 accsp, srcb, dstb, gi, wi, rb, zb, gsems, ssems, zsem):
    c = lax.axis_index("c")
    s = lax.axis_index("s")
    roff = c * N
    CH = 64
    NCHV = EPT // CH
    for r in range(L):
        for k in range(8):
            zb[r, pl.ds(L * k, L)] = jnp.zeros((L,), jnp.float32)
    for d in range(2):
        ytab = yin_ref if d == 0 else yout_ref

        for k0 in range(0, 40, 8):
            zd = [pltpu.async_copy(zb, accsp.at[pl.ds(640 * s + L * k, L)],
                                   zsem) for k in range(k0, k0 + 8)]
            for cp in zd:
                cp.wait()
        plsc.subcore_barrier()
        pltpu.sync_copy(srci_ref.at[d, pl.ds(EPT * s, EPT)], srcb)
        pltpu.sync_copy(dsti_ref.at[d, pl.ds(EPT * s, EPT)], dstb)

        def stage(jj, b):
            for k in range(CH // L):
                sl = pl.ds(L * k, L)
                gi[b][sl] = srcb[pl.ds(CH * jj + L * k, L)] + roff
                wi[b][sl] = dstb[pl.ds(CH * jj + L * k, L)]

        for b in range(4):
            stage(b, b)
            pltpu.async_copy(ytab.at[gi[b]], rb[b], gsems[b])

        @pl.loop(0, NCHV, step=4)
        def _(j):
            for b in range(4):
                jj = j + b
                pltpu.make_async_copy(ytab.at[gi[b]], rb[b],
                                      gsems[b]).wait()
                pltpu.async_copy(rb[b], accsp.at[wi[b]], ssems[b],
                                 add=True)
                pltpu.make_async_copy(rb[b], accsp.at[wi[b]],
                                      ssems[b]).wait()

                @pl.when(jj + 4 < NCHV)
                def _():
                    stage(jj + 4, b)
                    pltpu.async_copy(ytab.at[gi[b]], rb[b], gsems[b])

        plsc.subcore_barrier()
        od = {}
        for k in range(10):
            b = k % 4
            if k - 4 >= 0:
                od[k - 4].wait()
            r0 = 640 * s + 64 * k
            pltpu.sync_copy(accsp.at[pl.ds(r0, 64)], rb[b])
            od[k] = pltpu.async_copy(
                rb[b], acc4_ref.at[d, c, pl.ds(r0, 64)], gsems[b])
        for k in range(6, 10):
            od[k].wait()
        plsc.subcore_barrier()


def _sc_acc(yin2, yout2, srcg, dstw):
    return pl.kernel(
        _sc_acc_body,
        out_type=jax.ShapeDtypeStruct((2, NC, ACCR, 128), jnp.float32),
        mesh=_MESH,
        scratch_types=[
            pltpu.VMEM_SHARED((ACCR, 128), jnp.float32),
            pltpu.VMEM((EPT,), jnp.int32),
            pltpu.VMEM((EPT,), jnp.int32),
            [pltpu.VMEM((64,), jnp.int32) for _ in range(4)],
            [pltpu.VMEM((64,), jnp.int32) for _ in range(4)],
            [pltpu.VMEM((64, 128), jnp.float32) for _ in range(4)],
            pltpu.VMEM((L, 128), jnp.float32),
            [pltpu.SemaphoreType.DMA for _ in range(4)],
            [pltpu.SemaphoreType.DMA for _ in range(4)],
            pltpu.SemaphoreType.DMA,
        ],
    )(yin2, yout2, srcg, dstw)


def kernel(x, rels, edge_index, edge_type,
           w_in1, w_out1, w_loop1, w_rel1,
           w_in2, w_out2, w_loop2, w_rel2,
           loop_rel1, loop_rel2):
    ne = edge_index.shape[1] // 2
    src_in, dst_in = edge_index[0, :ne], edge_index[1, :ne]
    src_out, dst_out = edge_index[0, ne:], edge_index[1, ne:]
    et_in, et_out = edge_type[:ne], edge_type[ne:]

    npad = EP - ne
    pad0 = jnp.zeros((npad,), jnp.int32)
    padd = jnp.full((npad,), DUMN, jnp.int32)

    def lay(a_in, a_out, pad):
        return jnp.stack([jnp.concatenate([a_in, pad]),
                          jnp.concatenate([a_out, pad])]).reshape(2, NS, NCH, 128)

    srcg = lay(src_in, src_out, pad0)
    srcd = lay(src_in, src_out, padd)
    dstw = lay(dst_in, dst_out, padd)
    etp = lay(et_in, et_out, pad0)

    deg2 = _sc_deg(srcd).reshape(NC, NPAD)
    dinv2 = _tc_dinv(deg2)
    dinvT = dinv2[:, :N].T
    cfl = _sc_cprime(dstw, etp, srcd, dinv2.reshape(-1))

    def cq(d, q):
        b = (d * 4 + q) * CSPQ
        return lax.dynamic_slice(cfl, (b,), (CQUSE,)).reshape(QN, NREL)

    c_in = jnp.concatenate([cq(0, q) for q in range(4)], axis=0)
    c_out = jnp.concatenate([cq(1, q) for q in range(4)], axis=0)

    w3_1 = jnp.concatenate([w_in1, w_out1, w_loop1], axis=1)
    w3_2 = jnp.concatenate([w_in2, w_out2, w_loop2], axis=1)
    rel_all1 = jnp.concatenate([rels, loop_rel1], axis=0)

    def layer(xl, rel_all, w3, w_rel):
        y_in, y_out, xwl, rw3, rnext = _tc_prep(
            xl, w3, rel_all, w_rel, dinvT)
        acc4 = _sc_acc(y_in.reshape(2 * N, 128), y_out.reshape(2 * N, 128),
                       srcg.reshape(2, EP), dstw.reshape(2, EP))
        pre, stat = _tc_combine(acc4, c_in, c_out, rw3, xwl, dinvT)
        return _tc_bn(pre, stat), rnext

    x1, r1 = layer(x, rel_all1, w3_1, w_rel1)
    rel_all2 = jnp.concatenate([r1, loop_rel2], axis=0)
    x2, r2 = layer(x1, rel_all2, w3_2, w_rel2)
    return (x2, r2)
